# baseline (speedup 1.0000x reference)
import functools

import jax
import jax.numpy as jnp
from jax import lax
from jax.experimental import pallas as pl
from jax.experimental.pallas import tpu as pltpu
from jax.experimental.pallas import tpu_sc as plsc

N = 100000
NSEG = 512
NW = 32
CH = 3200
NP = NW * CH
NIDX = CH // 128
CHA = 2 * CH
NIDXA = CHA // 128
AROWS = 520
NROWS = 33

PADN = 303104
PROWS = PADN // 128
BROWS = PROWS // 8

SCALE_POS = 2.0 * 3.141592653589793
TIME_T = 2.0
EPS = 1e-05

_MESH = dict(core_axis_name="c", subcore_axis_name="s")
_PARAMS = pltpu.CompilerParams(use_tc_tiling_on_sc=False)


def _wid():
    return lax.axis_index("c") * 16 + lax.axis_index("s")


def _seg_sums_body(rec_hbm, idx_hbm, zero_hbm, means_hbm, recs, ridx, nbuf,
                   shared):
    sid = lax.axis_index("s")
    base = sid * CHA

    @pl.when(sid == 0)
    def _():
        pltpu.sync_copy(zero_hbm, shared)

    plsc.subcore_barrier()
    pltpu.sync_copy(rec_hbm.at[pl.ds(base, CHA), :], recs)
    pltpu.sync_copy(idx_hbm.at[pl.ds(sid * NIDXA, NIDXA), :], ridx)
    for j in range(NIDXA):
        pltpu.sync_copy(recs.at[pl.ds(j * 128, 128), :],
                        shared.at[ridx.at[j]], add=True)
    plsc.subcore_barrier()

    nb = jnp.minimum(sid * NROWS, AROWS - NROWS)
    pltpu.sync_copy(shared.at[pl.ds(nb, NROWS)], nbuf)

    @functools.partial(lax.fori_loop, 0, NROWS, init_val=None)
    def _(i, _):
        m = nbuf[i]
        cv = jnp.full((16,), jnp.maximum(m[6], 1.0), jnp.float32)
        nbuf[i] = m / cv

    pltpu.sync_copy(nbuf, means_hbm.at[pl.ds(nb, NROWS), :])


def _center_body(rec_hbm, idx_hbm, means_hbm, out_hbm, recs, mrows, ridx,
                 sem):
    w = _wid()
    base = w * CH
    pltpu.sync_copy(rec_hbm.at[pl.ds(base, CH), :], recs)
    pltpu.sync_copy(idx_hbm.at[pl.ds(w * NIDX, NIDX), :], ridx)
    descs = [pltpu.async_copy(means_hbm.at[ridx.at[j]],
                              mrows.at[pl.ds(j * 128, 128), :], sem)
             for j in range(NIDX)]
    for d in descs:
        d.wait()

    @functools.partial(lax.fori_loop, 0, CH, init_val=None)
    def _(i, _):
        recs[i] = recs[i] - mrows[i]

    pltpu.sync_copy(recs, out_hbm.at[pl.ds(base, CH), :])


def _dense_body(t_ref, f0_ref, v0_ref, evc_ref, erc_ref,
                ft_ref, vt_ref, rt_ref, evco_ref, erco_ref):
    inv = jnp.float32(1.0 / SCALE_POS)
    tt = TIME_T * jnp.broadcast_to(t_ref[...], (t_ref.shape[0], 3))
    ev_s = evc_ref[...] * inv
    er_s = erc_ref[...] * inv
    f0 = f0_ref[...]
    v0 = v0_ref[...]
    en = jnp.exp(-tt)
    d = 1.0 / (1.0 + en)
    sv = jnp.sqrt(jnp.clip(1.0 - en * en, EPS, None))
    vt = en * v0 + sv * ev_s
    pref = (1.0 - en) * d
    sr = jnp.sqrt(jnp.clip(2.0 * tt + 8.0 * en * d - 4.0, EPS, None))
    rt = jnp.remainder(pref * (vt + v0) + sr * er_s + 0.5,
                       jnp.float32(1.0)) - 0.5
    ft = jnp.remainder(f0 + rt + 0.5, jnp.float32(1.0)) - 0.5
    ft_ref[...] = ft
    vt_ref[...] = vt
    rt_ref[...] = rt
    evco_ref[...] = evc_ref[...]
    erco_ref[...] = erc_ref[...]


def _f32(shape):
    return jax.ShapeDtypeStruct(shape, jnp.float32)


_seg_sums = functools.partial(
    pl.kernel,
    out_type=_f32((AROWS, 16)),
    mesh=plsc.VectorSubcoreMesh(num_cores=1, **_MESH),
    compiler_params=_PARAMS,
    scratch_types=[
        pltpu.VMEM((CHA, 16), jnp.float32),
        pltpu.VMEM((NIDXA, 128), jnp.int32),
        pltpu.VMEM((NROWS, 16), jnp.float32),
        pltpu.VMEM_SHARED((AROWS, 16), jnp.float32),
    ],
)(_seg_sums_body)

_center = functools.partial(
    pl.kernel,
    out_type=_f32((NP, 16)),
    mesh=plsc.VectorSubcoreMesh(**_MESH),
    compiler_params=_PARAMS,
    scratch_types=[
        pltpu.VMEM((CH, 16), jnp.float32),
        pltpu.VMEM((CH, 16), jnp.float32),
        pltpu.VMEM((NIDX, 128), jnp.int32),
        pltpu.SemaphoreType.DMA,
    ],
)(_center_body)

DBR = N // 25

_dense = pl.pallas_call(
    _dense_body,
    grid=(25,),
    in_specs=[pl.BlockSpec((DBR, 1), lambda i: (i, 0))]
    + [pl.BlockSpec((DBR, 3), lambda i: (i, 0))] * 4,
    out_specs=[pl.BlockSpec((DBR, 3), lambda i: (i, 0))] * 5,
    out_shape=(_f32((N, 3)),) * 5,
)


def _pad2d(x):
    return jnp.pad(x, (0, PADN - 3 * N)).reshape(PROWS, 128)


def kernel(t, f0, index, v0, epsilon_v, epsilon_r):
    idx = index.astype(jnp.int32)

    records = jnp.concatenate(
        [epsilon_v, epsilon_r, jnp.ones((N, 1), jnp.float32),
         jnp.zeros((N, 9), jnp.float32)], axis=1)
    records = jnp.pad(records, ((0, NP - N), (0, 0)))
    idx_pad = jnp.pad(idx, (0, NP - N), constant_values=NSEG)
    idx2d = idx_pad.reshape(NP // 128, 128)
    zeros = jnp.zeros((AROWS, 16), jnp.float32)

    means = _seg_sums(records, idx2d, zeros)
    centered = _center(records, idx2d, means)

    evc = centered[:N, 0:3]
    erc = centered[:N, 3:6]

    ft, vt, rt, evco, erco = _dense(t.reshape(N, 1), f0, v0, evc, erc)
    return (ft, vt, evco, erco, rt)


# speedup vs baseline: 1.0464x; 1.0464x over previous
import functools

import jax
import jax.numpy as jnp
from jax import lax
from jax.experimental import pallas as pl
from jax.experimental.pallas import tpu as pltpu
from jax.experimental.pallas import tpu_sc as plsc

N = 100000
NSEG = 512
NW = 32
CH = 3200
NP = NW * CH
NIDX = CH // 128
CHA = 2 * CH
NIDXA = CHA // 128
AROWS = 520
NROWS = 33

PADN = 303104
PROWS = PADN // 128
BROWS = PROWS // 8

SCALE_POS = 2.0 * 3.141592653589793
TIME_T = 2.0
EPS = 1e-05

_MESH = dict(core_axis_name="c", subcore_axis_name="s")
_PARAMS = pltpu.CompilerParams(use_tc_tiling_on_sc=False)


def _wid():
    return lax.axis_index("c") * 16 + lax.axis_index("s")


def _seg_sums_body(rec_hbm, idx_hbm, zero_hbm, means_hbm, recs, ridx, nbuf,
                   shared):
    sid = lax.axis_index("s")
    base = sid * CHA

    @pl.when(sid == 0)
    def _():
        pltpu.sync_copy(zero_hbm, shared)

    plsc.subcore_barrier()
    pltpu.sync_copy(rec_hbm.at[pl.ds(base, CHA), :], recs)
    pltpu.sync_copy(idx_hbm.at[pl.ds(sid * NIDXA, NIDXA), :], ridx)
    for j in range(NIDXA):
        pltpu.sync_copy(recs.at[pl.ds(j * 128, 128), :],
                        shared.at[ridx.at[j]], add=True)
    plsc.subcore_barrier()

    nb = jnp.minimum(sid * NROWS, AROWS - NROWS)
    pltpu.sync_copy(shared.at[pl.ds(nb, NROWS)], nbuf)

    @functools.partial(lax.fori_loop, 0, NROWS, init_val=None)
    def _(i, _):
        m = nbuf[i]
        cv = jnp.full((16,), jnp.maximum(m[6], 1.0), jnp.float32)
        nbuf[i] = m / cv

    pltpu.sync_copy(nbuf, means_hbm.at[pl.ds(nb, NROWS), :])


def _center_body(rec_hbm, idx_hbm, means_hbm, out_hbm, recs, mrows, ridx,
                 sem):
    w = _wid()
    base = w * CH
    pltpu.sync_copy(rec_hbm.at[pl.ds(base, CH), :], recs)
    pltpu.sync_copy(idx_hbm.at[pl.ds(w * NIDX, NIDX), :], ridx)
    descs = [pltpu.async_copy(means_hbm.at[ridx.at[j]],
                              mrows.at[pl.ds(j * 128, 128), :], sem)
             for j in range(NIDX)]
    for d in descs:
        d.wait()

    @functools.partial(lax.fori_loop, 0, CH, init_val=None)
    def _(i, _):
        recs[i] = recs[i] - mrows[i]

    pltpu.sync_copy(recs, out_hbm.at[pl.ds(base, CH), :])


def _dense_body(t_ref, f0_ref, v0_ref, evc_ref, erc_ref,
                ft_ref, vt_ref, rt_ref):
    inv = jnp.float32(1.0 / SCALE_POS)
    tt = TIME_T * jnp.broadcast_to(t_ref[...], (t_ref.shape[0], 3))
    ev_s = evc_ref[...] * inv
    er_s = erc_ref[...] * inv
    f0 = f0_ref[...]
    v0 = v0_ref[...]
    en = jnp.exp(-tt)
    d = 1.0 / (1.0 + en)
    sv = jnp.sqrt(jnp.clip(1.0 - en * en, EPS, None))
    vt = en * v0 + sv * ev_s
    pref = (1.0 - en) * d
    sr = jnp.sqrt(jnp.clip(2.0 * tt + 8.0 * en * d - 4.0, EPS, None))
    rt = jnp.remainder(pref * (vt + v0) + sr * er_s + 0.5,
                       jnp.float32(1.0)) - 0.5
    ft = jnp.remainder(f0 + rt + 0.5, jnp.float32(1.0)) - 0.5
    ft_ref[...] = ft
    vt_ref[...] = vt
    rt_ref[...] = rt


def _f32(shape):
    return jax.ShapeDtypeStruct(shape, jnp.float32)


_seg_sums = functools.partial(
    pl.kernel,
    out_type=_f32((AROWS, 16)),
    mesh=plsc.VectorSubcoreMesh(num_cores=1, **_MESH),
    compiler_params=_PARAMS,
    scratch_types=[
        pltpu.VMEM((CHA, 16), jnp.float32),
        pltpu.VMEM((NIDXA, 128), jnp.int32),
        pltpu.VMEM((NROWS, 16), jnp.float32),
        pltpu.VMEM_SHARED((AROWS, 16), jnp.float32),
    ],
)(_seg_sums_body)

_center = functools.partial(
    pl.kernel,
    out_type=_f32((NP, 16)),
    mesh=plsc.VectorSubcoreMesh(**_MESH),
    compiler_params=_PARAMS,
    scratch_types=[
        pltpu.VMEM((CH, 16), jnp.float32),
        pltpu.VMEM((CH, 16), jnp.float32),
        pltpu.VMEM((NIDX, 128), jnp.int32),
        pltpu.SemaphoreType.DMA,
    ],
)(_center_body)

DBR = N // 25

_dense = pl.pallas_call(
    _dense_body,
    grid=(25,),
    in_specs=[pl.BlockSpec((DBR, 1), lambda i: (i, 0))]
    + [pl.BlockSpec((DBR, 3), lambda i: (i, 0))] * 4,
    out_specs=[pl.BlockSpec((DBR, 3), lambda i: (i, 0))] * 3,
    out_shape=(_f32((N, 3)),) * 3,
)


def _pad2d(x):
    return jnp.pad(x, (0, PADN - 3 * N)).reshape(PROWS, 128)


def kernel(t, f0, index, v0, epsilon_v, epsilon_r):
    idx = index.astype(jnp.int32)

    records = jnp.concatenate(
        [epsilon_v, epsilon_r, jnp.ones((N, 1), jnp.float32),
         jnp.zeros((N, 9), jnp.float32)], axis=1)
    records = jnp.pad(records, ((0, NP - N), (0, 0)))
    idx_pad = jnp.pad(idx, (0, NP - N), constant_values=NSEG)
    idx2d = idx_pad.reshape(NP // 128, 128)
    zeros = jnp.zeros((AROWS, 16), jnp.float32)

    means = _seg_sums(records, idx2d, zeros)
    centered = _center(records, idx2d, means)

    evc = centered[:N, 0:3]
    erc = centered[:N, 3:6]

    ft, vt, rt = _dense(t.reshape(N, 1), f0, v0, evc, erc)
    return (ft, vt, evc, erc, rt)
